# baseline (speedup 1.0000x reference)
import jax
import jax.numpy as jnp
from jax.experimental import pallas as pl
from jax.experimental.pallas import tpu as pltpu

_TOP_K = 8
_SQRT_HALF = 0.7071067811865476
_LN_EPS = 1e-5


def _epilogue(acc, b1, gamma, beta, w2, b2):
    h = acc + b1
    mu = jnp.mean(h, axis=-1, keepdims=True)
    xc = h - mu
    var = jnp.mean(xc * xc, axis=-1, keepdims=True)
    h = xc * jax.lax.rsqrt(var + _LN_EPS) * gamma + beta
    h = 0.5 * h * (1.0 + jax.lax.erf(h * _SQRT_HALF))
    logits = jnp.dot(h, w2, preferred_element_type=jnp.float32) + b2

    e_dim = logits.shape[-1]
    iota_e = jax.lax.broadcasted_iota(jnp.int32, logits.shape, 1)
    neg_inf = jnp.float32(-jnp.inf)
    alive = logits == logits
    sel = jnp.zeros(logits.shape, jnp.bool_)
    m0 = None
    for r in range(_TOP_K):
        lm = jnp.where(alive, logits, neg_inf)
        m = jnp.max(lm, axis=-1, keepdims=True)
        if r == 0:
            m0 = m
        eq = lm == m
        first = jnp.min(jnp.where(eq, iota_e, e_dim), axis=-1, keepdims=True)
        new = eq & (iota_e == first)
        sel = sel | new
        alive = alive & (~new)

    ex = jnp.where(sel, jnp.exp(logits - m0), 0.0)
    s = jnp.sum(ex, axis=-1, keepdims=True)
    return ex / s


def _make_body(nk, bm, ec):
    nchunk = bm // ec

    def _body(x_ref, w1_ref, b1_ref, g_ref, bt_ref, w2_ref, b2_ref, o_ref, acc_ref):
        k = pl.program_id(1)

        @pl.when(k == 0)
        def _init():
            acc_ref[...] = jnp.zeros_like(acc_ref)

        xb = x_ref[...]
        if xb.dtype != jnp.bfloat16:
            xb = xb.astype(jnp.bfloat16)
        wb = w1_ref[...].astype(jnp.bfloat16)
        acc_ref[...] += jnp.dot(xb, wb, preferred_element_type=jnp.float32)

        @pl.when(k == nk - 1)
        def _fin():
            def chunk(cc, carry):
                row = cc * ec
                o_ref[pl.ds(row, ec), :] = _epilogue(
                    acc_ref[pl.ds(row, ec), :],
                    b1_ref[...],
                    g_ref[...],
                    bt_ref[...],
                    w2_ref[...],
                    b2_ref[...],
                )
                return carry

            jax.lax.fori_loop(0, nchunk, chunk, 0)

    return _body


def _router_block(flat, W1, b1r, gr, btr, W2, b2r):
    rows, kdim = flat.shape
    d_out = W1.shape[1]
    e_dim = W2.shape[1]
    bm = min(2048, rows)
    nb = rows // bm
    bk = 512
    nk = kdim // bk
    ec = min(512, bm)

    return pl.pallas_call(
        _make_body(nk, bm, ec),
        grid=(nb, nk),
        in_specs=[
            pl.BlockSpec((bm, bk), lambda i, k: (i, k)),
            pl.BlockSpec((bk, d_out), lambda i, k: (k, 0)),
            pl.BlockSpec((1, d_out), lambda i, k: (0, 0)),
            pl.BlockSpec((1, d_out), lambda i, k: (0, 0)),
            pl.BlockSpec((1, d_out), lambda i, k: (0, 0)),
            pl.BlockSpec((d_out, e_dim), lambda i, k: (0, 0)),
            pl.BlockSpec((1, e_dim), lambda i, k: (0, 0)),
        ],
        out_specs=pl.BlockSpec((bm, e_dim), lambda i, k: (i, 0)),
        out_shape=jax.ShapeDtypeStruct((rows, e_dim), jnp.float32),
        scratch_shapes=[pltpu.VMEM((bm, d_out), jnp.float32)],
        compiler_params=pltpu.CompilerParams(
            dimension_semantics=("parallel", "arbitrary")
        ),
    )(flat, W1, b1r, gr, btr, W2, b2r)


def _repack_body(x_ref, o_ref):
    d = x_ref.shape[2]
    for c in range(x_ref.shape[1]):
        o_ref[:, c * d : (c + 1) * d] = x_ref[:, c, :].astype(jnp.bfloat16)


def _repack(triplet):
    b_dim, three, d_in = triplet.shape
    kdim = three * d_in
    bm = min(128, b_dim)
    nb = b_dim // bm
    return pl.pallas_call(
        _repack_body,
        grid=(nb,),
        in_specs=[pl.BlockSpec((bm, three, d_in), lambda i: (i, 0, 0))],
        out_specs=pl.BlockSpec((bm, kdim), lambda i: (i, 0)),
        out_shape=jax.ShapeDtypeStruct((b_dim, kdim), jnp.bfloat16),
        compiler_params=pltpu.CompilerParams(dimension_semantics=("arbitrary",)),
    )(triplet)


def kernel(triplet, W1, b1, gamma, beta, W2, b2):
    b_dim, three, d_in = triplet.shape
    d_out = W1.shape[1]
    e_dim = W2.shape[1]

    b1r = b1.reshape(1, d_out)
    gr = gamma.reshape(1, d_out)
    btr = beta.reshape(1, d_out)
    b2r = b2.reshape(1, e_dim)

    flat = _repack(triplet)
    return _router_block(flat, W1, b1r, gr, btr, W2, b2r)


# speedup vs baseline: 1.2824x; 1.2824x over previous
import jax
import jax.numpy as jnp
from jax.experimental import pallas as pl
from jax.experimental.pallas import tpu as pltpu

_TOP_K = 8
_SQRT_HALF = 0.7071067811865476
_LN_EPS = 1e-5


def _epilogue(acc, b1, gamma, beta, w2, b2):
    h = acc + b1
    mu = jnp.mean(h, axis=-1, keepdims=True)
    xc = h - mu
    var = jnp.mean(xc * xc, axis=-1, keepdims=True)
    h = xc * jax.lax.rsqrt(var + _LN_EPS) * gamma + beta
    h = 0.5 * h * (1.0 + jax.lax.erf(h * _SQRT_HALF))
    logits = jnp.dot(h, w2, preferred_element_type=jnp.float32) + b2

    e_dim = logits.shape[-1]
    iota_e = jax.lax.broadcasted_iota(jnp.int32, logits.shape, 1)
    neg_inf = jnp.float32(-jnp.inf)
    alive = logits == logits
    sel = jnp.zeros(logits.shape, jnp.bool_)
    m0 = None
    for r in range(_TOP_K):
        lm = jnp.where(alive, logits, neg_inf)
        m = jnp.max(lm, axis=-1, keepdims=True)
        if r == 0:
            m0 = m
        eq = lm == m
        first = jnp.min(jnp.where(eq, iota_e, e_dim), axis=-1, keepdims=True)
        new = eq & (iota_e == first)
        sel = sel | new
        alive = alive & (~new)

    ex = jnp.where(sel, jnp.exp(logits - m0), 0.0)
    s = jnp.sum(ex, axis=-1, keepdims=True)
    return ex / s


def _make_body(nk, bm, ec):
    nchunk = bm // ec

    def _body(x_ref, w1_ref, b1_ref, g_ref, bt_ref, w2_ref, b2_ref, o_ref, acc_ref):
        k = pl.program_id(1)

        @pl.when(k == 0)
        def _init():
            acc_ref[...] = jnp.zeros_like(acc_ref)

        xb = x_ref[...]
        if xb.dtype != jnp.bfloat16:
            xb = xb.astype(jnp.bfloat16)
        wb = w1_ref[...].astype(jnp.bfloat16)
        acc_ref[...] += jnp.dot(xb, wb, preferred_element_type=jnp.float32)

        @pl.when(k == nk - 1)
        def _fin():
            def chunk(cc, carry):
                row = cc * ec
                o_ref[pl.ds(row, ec), :] = _epilogue(
                    acc_ref[pl.ds(row, ec), :],
                    b1_ref[...],
                    g_ref[...],
                    bt_ref[...],
                    w2_ref[...],
                    b2_ref[...],
                )
                return carry

            jax.lax.fori_loop(0, nchunk, chunk, 0)

    return _body


def _router_block(flat, W1, b1r, gr, btr, W2, b2r):
    rows, kdim = flat.shape
    d_out = W1.shape[1]
    e_dim = W2.shape[1]
    bm = min(1024, rows)
    nb = rows // bm
    bk = 1024
    nk = kdim // bk
    ec = min(512, bm)

    return pl.pallas_call(
        _make_body(nk, bm, ec),
        grid=(nb, nk),
        in_specs=[
            pl.BlockSpec((bm, bk), lambda i, k: (i, k)),
            pl.BlockSpec((bk, d_out), lambda i, k: (k, 0)),
            pl.BlockSpec((1, d_out), lambda i, k: (0, 0)),
            pl.BlockSpec((1, d_out), lambda i, k: (0, 0)),
            pl.BlockSpec((1, d_out), lambda i, k: (0, 0)),
            pl.BlockSpec((d_out, e_dim), lambda i, k: (0, 0)),
            pl.BlockSpec((1, e_dim), lambda i, k: (0, 0)),
        ],
        out_specs=pl.BlockSpec((bm, e_dim), lambda i, k: (i, 0)),
        out_shape=jax.ShapeDtypeStruct((rows, e_dim), jnp.float32),
        scratch_shapes=[pltpu.VMEM((bm, d_out), jnp.float32)],
        compiler_params=pltpu.CompilerParams(
            dimension_semantics=("parallel", "arbitrary")
        ),
    )(flat, W1, b1r, gr, btr, W2, b2r)


def kernel(triplet, W1, b1, gamma, beta, W2, b2):
    b_dim, three, d_in = triplet.shape
    kdim = three * d_in
    d_out = W1.shape[1]
    e_dim = W2.shape[1]

    b1r = b1.reshape(1, d_out)
    gr = gamma.reshape(1, d_out)
    btr = beta.reshape(1, d_out)
    b2r = b2.reshape(1, e_dim)

    flat = triplet.reshape(b_dim, kdim).astype(jnp.bfloat16)
    return _router_block(flat, W1, b1r, gr, btr, W2, b2r)
